# baseline (speedup 1.0000x reference)
import functools

import jax
import jax.numpy as jnp
from jax import lax
from jax.experimental import pallas as pl
from jax.experimental.pallas import tpu as pltpu
from jax.experimental.pallas import tpu_sc as plsc

_N = 10000
_E = 320000
_D = 128
_H = 128
_B = 256
_S = 16

_NC = 2
_NS = 16
_NW = _NC * _NS
_K = 128
_NCHUNK = 80
_EPAD = _NW * _NCHUNK * _K
_GB = 8
_NG = _NCHUNK // _GB
_NGT = 2 * _NG
_G0 = 14
_NP = 10240
_RPS = _NP // _NS
_C0 = 144

_TPB = (_B * _S) // _NW
_SEG = _TPB // _S
_FPT = 640


def _sc_mesh():
    return plsc.VectorSubcoreMesh(core_axis_name="c", subcore_axis_name="s",
                                  num_cores=_NC, num_subcores=_NS)


_SC_PARAMS = pltpu.CompilerParams(needs_layout_passes=False)


@functools.cache
def _build_prep_kernel():
    return functools.partial(
        pl.kernel,
        out_type=jax.ShapeDtypeStruct((_NW + 1, _NP), jnp.float32),
        mesh=_sc_mesh(),
        scratch_types=[
            pltpu.VMEM((_B, _S), jnp.int32),
            pltpu.VMEM((_NCHUNK, _K), jnp.int32),
            pltpu.VMEM((_NCHUNK, _K), jnp.int32),
            pltpu.VMEM((_NCHUNK, _K), jnp.float32),
            pltpu.VMEM((1, _NP), jnp.float32),
            pltpu.VMEM((1, _NP), jnp.float32),
        ],
        compiler_params=_SC_PARAMS,
    )(_prep_body)


def _prep_body(bat_hbm, src_hbm, dst_hbm, ew_hbm, aggf_hbm,
               bat_v, src_v, dst_v, ew_v, f_v, aggf_v):
    c = lax.axis_index("c")
    s = lax.axis_index("s")
    w = s * _NC + c
    pltpu.sync_copy(bat_hbm, bat_v)
    pltpu.sync_copy(src_hbm.at[w], src_v)
    pltpu.sync_copy(dst_hbm.at[w], dst_v)
    pltpu.sync_copy(ew_hbm.at[w], ew_v)
    zeros16 = jnp.zeros((16,), jnp.float32)

    def zero_body(i, _):
        f_v[0, pl.ds(i * 16, 16)] = zeros16
        aggf_v[0, pl.ds(i * 16, 16)] = zeros16
        return 0

    lax.fori_loop(0, _NP // 16, zero_body, 0)
    ones = jnp.ones((16,), jnp.float32)
    z16 = jnp.zeros((16,), jnp.int32)

    def scat_body(i, _):
        plsc.store_scatter(f_v, [z16, bat_v[i, :]], ones)
        return 0

    lax.fori_loop(0, _B, scat_body, 0)

    def grp_body(i, _):
        j = i // (_K // 16)
        g = i % (_K // 16)
        sl = pl.ds(g * 16, 16)
        sv = src_v[j, sl]
        dv = dst_v[j, sl]
        ev = ew_v[j, sl]
        fv = plsc.load_gather(f_v, [z16, sv])
        plsc.addupdate_scatter(aggf_v, [z16, dv], fv * ev)
        return 0

    lax.fori_loop(0, _NCHUNK * (_K // 16), grp_body, 0)
    pltpu.sync_copy(aggf_v, aggf_hbm.at[pl.ds(w, 1)])

    @pl.when(jnp.logical_and(c == 0, s == 0))
    def _():
        pltpu.sync_copy(f_v, aggf_hbm.at[pl.ds(_NW, 1)])


@functools.cache
def _build_agg_kernel():
    @functools.partial(
        pl.kernel,
        out_type=jax.ShapeDtypeStruct((_NC, _NP, _H), jnp.float32),
        mesh=_sc_mesh(),
        scratch_types=[
            pltpu.VMEM((2 * _GB, _K), jnp.int32),
            pltpu.VMEM((2 * _GB, _K), jnp.int32),
            pltpu.VMEM((2 * _GB, _K), jnp.float32),
            pltpu.VMEM((_K, _H), jnp.float32),
            pltpu.VMEM((_K, _H), jnp.float32),
            pltpu.VMEM_SHARED((_NP, _H), jnp.float32),
            pltpu.SemaphoreType.DMA,
            pltpu.SemaphoreType.DMA,
            pltpu.SemaphoreType.DMA,
        ],
        compiler_params=_SC_PARAMS,
    )
    def agg(h_hbm, src_hbm, dst_hbm, ew_hbm, out_hbm,
            src_v, dst_v, ew_v, rows0, rows1, acc, sg0, sg1, si):
        c = lax.axis_index("c")
        s = lax.axis_index("s")
        ng = jnp.where(c == 0, _G0, _NGT - _G0)
        nchunk = ng * _GB
        wg = s * _NGT + jnp.where(c == 0, 0, _G0)
        zeros16 = jnp.zeros((16,), jnp.float32)

        def zbody(i, _):
            rows0[i // (_H // 16), pl.ds((i % (_H // 16)) * 16, 16)] = zeros16
            return 0

        lax.fori_loop(0, _K * (_H // 16), zbody, 0)
        for zz in range(_RPS // _K):
            pltpu.sync_copy(rows0, acc.at[pl.ds(s * _RPS + zz * _K, _K)])
        pltpu.sync_copy(src_hbm.at[wg], src_v.at[pl.ds(0, _GB)])
        pltpu.sync_copy(dst_hbm.at[wg], dst_v.at[pl.ds(0, _GB)])
        pltpu.sync_copy(ew_hbm.at[wg], ew_v.at[pl.ds(0, _GB)])
        plsc.subcore_barrier()
        pltpu.async_copy(h_hbm.at[src_v.at[0]], rows0, sg0)

        def scale(rows_ref, row):
            rf = jnp.full((16,), row, jnp.int32)

            def sbody(r, _):
                ewb = plsc.load_gather(
                    ew_v, [rf, jnp.full((16,), r, jnp.int32)])
                for kk in range(_H // 16):
                    sl = pl.ds(kk * 16, 16)
                    rows_ref[r, sl] = rows_ref[r, sl] * ewb
                return 0

            lax.fori_loop(0, _K, sbody, 0)

        def chunk(g, p, jj, b):
            j = g * _GB + jj
            row = p * _GB + jj
            sg_self, sg_other = (sg0, sg1) if b == 0 else (sg1, sg0)
            rows_self, rows_other = (rows0, rows1) if b == 0 else (rows1, rows0)
            pltpu.make_async_copy(h_hbm.at[src_v.at[row]],
                                  rows_self, sg_self).wait()

            @pl.when(j + 1 < nchunk)
            def _():
                @pl.when(jj + 1 < _GB)
                def _():
                    pltpu.async_copy(h_hbm.at[src_v.at[row + 1]],
                                     rows_other, sg_other)

                @pl.when(jj + 1 == _GB)
                def _():
                    q = 1 - p
                    pltpu.make_async_copy(src_hbm.at[wg],
                                          src_v.at[pl.ds(0, _GB)], si).wait()
                    pltpu.make_async_copy(dst_hbm.at[wg],
                                          dst_v.at[pl.ds(0, _GB)], si).wait()
                    pltpu.make_async_copy(ew_hbm.at[wg],
                                          ew_v.at[pl.ds(0, _GB)], si).wait()
                    pltpu.async_copy(h_hbm.at[src_v.at[q * _GB]],
                                     rows_other, sg_other)

            scale(rows_self, row)
            pltpu.sync_copy(rows_self, acc.at[dst_v.at[row]], add=True)

        def pair_body(t, g, p):
            @pl.when(jnp.logical_and(t == 1, g + 1 < ng))
            def _():
                q = 1 - p
                qs = pl.ds(q * _GB, _GB)
                pltpu.async_copy(src_hbm.at[wg + g + 1], src_v.at[qs], si)
                pltpu.async_copy(dst_hbm.at[wg + g + 1], dst_v.at[qs], si)
                pltpu.async_copy(ew_hbm.at[wg + g + 1], ew_v.at[qs], si)

            chunk(g, p, 2 * t, 0)
            chunk(g, p, 2 * t + 1, 1)

        def group_body(g, _):
            p = g % 2

            def inner(t, _):
                pair_body(t, g, p)
                return 0

            lax.fori_loop(0, _GB // 2, inner, 0)
            return 0

        lax.fori_loop(0, ng, group_body, 0)
        plsc.subcore_barrier()
        pltpu.sync_copy(acc.at[pl.ds(s * _RPS, _RPS)],
                        out_hbm.at[c, pl.ds(s * _RPS, _RPS)])

    return agg


@functools.cache
def _build_pool_kernel():
    return functools.partial(
        pl.kernel,
        out_type=jax.ShapeDtypeStruct((_B, 3 * _H), jnp.float32),
        mesh=_sc_mesh(),
        scratch_types=[
            pltpu.VMEM((_TPB,), jnp.int32),
            pltpu.VMEM((_TPB, _H), jnp.float32),
            pltpu.VMEM((_SEG, 3 * _H), jnp.float32),
            pltpu.SemaphoreType.DMA,
        ],
        compiler_params=_SC_PARAMS,
    )(_pool_body)


def _pool_body(h_hbm, bat_hbm, z_hbm, idx_v, rows_v, out_v, sem):
    c = lax.axis_index("c")
    s = lax.axis_index("s")
    w = s * _NC + c
    pltpu.sync_copy(bat_hbm.at[w], idx_v)
    pltpu.async_copy(h_hbm.at[idx_v], rows_v, sem).wait()
    inv_s = jnp.float32(1.0 / _S)
    for t in range(_SEG):
        for kk in range(_H // 16):
            sl = pl.ds(kk * 16, 16)
            v0 = rows_v[t * _S, sl]

            def red_body(rr, carry):
                sm, mn, mx = carry
                v = rows_v[t * _S + rr, sl]
                return (sm + v, jnp.minimum(mn, v), jnp.maximum(mx, v))

            sm, mn, mx = lax.fori_loop(1, _S, red_body, (v0, v0, v0))
            out_v[t, pl.ds(kk * 16, 16)] = sm * inv_s
            out_v[t, pl.ds(_H + kk * 16, 16)] = mn
            out_v[t, pl.ds(2 * _H + kk * 16, 16)] = mx
    pltpu.sync_copy(out_v, z_hbm.at[pl.ds(w * _SEG, _SEG)])


_BR = 1024


def _build_dense_layer(with_f):
    def body(*refs):
        if with_f:
            h_ref, a0_ref, a1_ref, af_ref, w_ref, wf_ref, b_ref, o_ref = refs
        else:
            h_ref, a0_ref, a1_ref, w_ref, b_ref, o_ref = refs
        acc = h_ref[...] + a0_ref[...] + a1_ref[...]
        out = jnp.dot(acc, w_ref[...], preferred_element_type=jnp.float32)
        if with_f:
            fsum = jnp.sum(af_ref[...], axis=0)
            out = out + fsum[:, None] * wf_ref[...]
        o_ref[...] = jnp.maximum(out + b_ref[...], 0.0)

    row_specs = [pl.BlockSpec((_BR, _H), lambda i: (i, 0))] * 3
    if with_f:
        in_specs = row_specs + [
            pl.BlockSpec((_NW + 1, _BR), lambda i: (0, i)),
            pl.BlockSpec((_H, _H), lambda i: (0, 0)),
            pl.BlockSpec((1, _H), lambda i: (0, 0)),
            pl.BlockSpec((1, _H), lambda i: (0, 0)),
        ]
    else:
        in_specs = row_specs + [
            pl.BlockSpec((_H, _H), lambda i: (0, 0)),
            pl.BlockSpec((1, _H), lambda i: (0, 0)),
        ]
    return pl.pallas_call(
        body,
        grid=(_NP // _BR,),
        in_specs=in_specs,
        out_specs=pl.BlockSpec((_BR, _H), lambda i: (i, 0)),
        out_shape=jax.ShapeDtypeStruct((_NP, _H), jnp.float32),
    )


_dense0 = _build_dense_layer(True)
_dense = _build_dense_layer(False)


def _ln(v, g, b, eps=1e-5):
    mu = jnp.mean(v, axis=-1, keepdims=True)
    var = jnp.mean((v - mu) ** 2, axis=-1, keepdims=True)
    return (v - mu) / jnp.sqrt(var + eps) * g + b


def _readout_body(z_ref, y_ref, r0w, r0b, g1r, be1r, r1w, r1b, g2r, be2r,
                  r2wt, r2b, o_ref):
    z = jnp.dot(z_ref[...], r0w[...], preferred_element_type=jnp.float32)
    z = jnp.maximum(z + r0b[...], 0.0)
    z = _ln(z, g1r[...], be1r[...])
    z = jnp.dot(z, r1w[...], preferred_element_type=jnp.float32) + r1b[...]
    z = _ln(z, g2r[...], be2r[...])
    z = jnp.maximum(z, 0.0)
    logits = jnp.sum(z * r2wt[...], axis=1, keepdims=True) + r2b[...]
    y = y_ref[...]
    loss = (jnp.maximum(logits, 0.0) - logits * y
            + jnp.log(1.0 + jnp.exp(-jnp.abs(logits))))
    o_ref[...] = jnp.mean(loss).reshape(1, 1)


_readout = pl.pallas_call(
    _readout_body,
    out_shape=jax.ShapeDtypeStruct((1, 1), jnp.float32),
)


def kernel(x, ei, ew, batches, labels, W0, b0, W1, b1, W2, b2,
           R0W, R0b, g1, be1, R1W, R1b, g2, be2, R2W, R2b):
    src = ei[0].astype(jnp.int32)
    dst = ei[1].astype(jnp.int32)
    pad = _EPAD - _E
    srcp = jnp.concatenate([src, jnp.zeros((pad,), jnp.int32)])
    dstp = jnp.concatenate([dst, jnp.zeros((pad,), jnp.int32)])
    ewp = jnp.concatenate([ew, jnp.zeros((pad,), jnp.float32)])
    srcp = srcp.reshape(_NW * _NG, _GB, _K)
    dstp = dstp.reshape(_NW * _NG, _GB, _K)
    ewp = ewp.reshape(_NW * _NG, _GB, _K)

    xp = jnp.concatenate([x, jnp.zeros((_NP - _N, _D), jnp.float32)], axis=0)

    af = _build_prep_kernel()(batches.astype(jnp.int32),
                              srcp.reshape(_NW, _NCHUNK, _K),
                              dstp.reshape(_NW, _NCHUNK, _K),
                              ewp.reshape(_NW, _NCHUNK, _K))
    agg = _build_agg_kernel()
    a0 = agg(xp, srcp, dstp, ewp)
    h1 = _dense0(xp, a0[0], a0[1], af, W0[:_D], W0[_D:_D + 1],
                 b0.reshape(1, _H))
    a1 = agg(h1, srcp, dstp, ewp)
    h2 = _dense(h1, a1[0], a1[1], W1, b1.reshape(1, _H))
    a2 = agg(h2, srcp, dstp, ewp)
    h3 = _dense(h2, a2[0], a2[1], W2, b2.reshape(1, _H))

    z = _build_pool_kernel()(h3, batches.astype(jnp.int32).reshape(_NW, _TPB))

    loss = _readout(
        z, labels.astype(jnp.float32), R0W, R0b.reshape(1, _H),
        g1.reshape(1, _H), be1.reshape(1, _H), R1W, R1b.reshape(1, _H),
        g2.reshape(1, _H), be2.reshape(1, _H), R2W.reshape(1, _H),
        R2b.reshape(1, 1))
    return loss[0, 0]


# speedup vs baseline: 1.0022x; 1.0022x over previous
import functools

import jax
import jax.numpy as jnp
from jax import lax
from jax.experimental import pallas as pl
from jax.experimental.pallas import tpu as pltpu
from jax.experimental.pallas import tpu_sc as plsc

_N = 10000
_E = 320000
_D = 128
_H = 128
_B = 256
_S = 16

_NC = 2
_NS = 16
_NW = _NC * _NS
_K = 128
_NCHUNK = 80
_EPAD = _NW * _NCHUNK * _K
_GB = 8
_NG = _NCHUNK // _GB
_NGT = 2 * _NG
_G0 = 15
_NP = 10240
_RPS = _NP // _NS
_C0 = 144

_TPB = (_B * _S) // _NW
_SEG = _TPB // _S
_FPT = 640


def _sc_mesh():
    return plsc.VectorSubcoreMesh(core_axis_name="c", subcore_axis_name="s",
                                  num_cores=_NC, num_subcores=_NS)


_SC_PARAMS = pltpu.CompilerParams(needs_layout_passes=False)


@functools.cache
def _build_prep_kernel():
    return functools.partial(
        pl.kernel,
        out_type=jax.ShapeDtypeStruct((_NW + 1, _NP), jnp.float32),
        mesh=_sc_mesh(),
        scratch_types=[
            pltpu.VMEM((_B, _S), jnp.int32),
            pltpu.VMEM((_NCHUNK, _K), jnp.int32),
            pltpu.VMEM((_NCHUNK, _K), jnp.int32),
            pltpu.VMEM((_NCHUNK, _K), jnp.float32),
            pltpu.VMEM((1, _NP), jnp.float32),
            pltpu.VMEM((1, _NP), jnp.float32),
        ],
        compiler_params=_SC_PARAMS,
    )(_prep_body)


def _prep_body(bat_hbm, src_hbm, dst_hbm, ew_hbm, aggf_hbm,
               bat_v, src_v, dst_v, ew_v, f_v, aggf_v):
    c = lax.axis_index("c")
    s = lax.axis_index("s")
    w = s * _NC + c
    pltpu.sync_copy(bat_hbm, bat_v)
    pltpu.sync_copy(src_hbm.at[w], src_v)
    pltpu.sync_copy(dst_hbm.at[w], dst_v)
    pltpu.sync_copy(ew_hbm.at[w], ew_v)
    zeros16 = jnp.zeros((16,), jnp.float32)

    def zero_body(i, _):
        f_v[0, pl.ds(i * 16, 16)] = zeros16
        aggf_v[0, pl.ds(i * 16, 16)] = zeros16
        return 0

    lax.fori_loop(0, _NP // 16, zero_body, 0)
    ones = jnp.ones((16,), jnp.float32)
    z16 = jnp.zeros((16,), jnp.int32)

    def scat_body(i, _):
        plsc.store_scatter(f_v, [z16, bat_v[i, :]], ones)
        return 0

    lax.fori_loop(0, _B, scat_body, 0)

    def grp_body(i, _):
        j = i // (_K // 16)
        g = i % (_K // 16)
        sl = pl.ds(g * 16, 16)
        sv = src_v[j, sl]
        dv = dst_v[j, sl]
        ev = ew_v[j, sl]
        fv = plsc.load_gather(f_v, [z16, sv])
        plsc.addupdate_scatter(aggf_v, [z16, dv], fv * ev)
        return 0

    lax.fori_loop(0, _NCHUNK * (_K // 16), grp_body, 0)
    pltpu.sync_copy(aggf_v, aggf_hbm.at[pl.ds(w, 1)])

    @pl.when(jnp.logical_and(c == 0, s == 0))
    def _():
        pltpu.sync_copy(f_v, aggf_hbm.at[pl.ds(_NW, 1)])


@functools.cache
def _build_agg_kernel():
    @functools.partial(
        pl.kernel,
        out_type=jax.ShapeDtypeStruct((_NC, _NP, _H), jnp.float32),
        mesh=_sc_mesh(),
        scratch_types=[
            pltpu.VMEM((2 * _GB, _K), jnp.int32),
            pltpu.VMEM((2 * _GB, _K), jnp.int32),
            pltpu.VMEM((2 * _GB, _K), jnp.float32),
            pltpu.VMEM((_K, _H), jnp.float32),
            pltpu.VMEM((_K, _H), jnp.float32),
            pltpu.VMEM_SHARED((_NP, _H), jnp.float32),
            pltpu.SemaphoreType.DMA,
            pltpu.SemaphoreType.DMA,
            pltpu.SemaphoreType.DMA,
        ],
        compiler_params=_SC_PARAMS,
    )
    def agg(h_hbm, src_hbm, dst_hbm, ew_hbm, out_hbm,
            src_v, dst_v, ew_v, rows0, rows1, acc, sg0, sg1, si):
        c = lax.axis_index("c")
        s = lax.axis_index("s")
        ng = jnp.where(c == 0, _G0, _NGT - _G0)
        nchunk = ng * _GB
        wg = s * _NGT + jnp.where(c == 0, 0, _G0)
        zeros16 = jnp.zeros((16,), jnp.float32)

        def zbody(i, _):
            rows0[i // (_H // 16), pl.ds((i % (_H // 16)) * 16, 16)] = zeros16
            return 0

        lax.fori_loop(0, _K * (_H // 16), zbody, 0)
        for zz in range(_RPS // _K):
            pltpu.sync_copy(rows0, acc.at[pl.ds(s * _RPS + zz * _K, _K)])
        pltpu.sync_copy(src_hbm.at[wg], src_v.at[pl.ds(0, _GB)])
        pltpu.sync_copy(dst_hbm.at[wg], dst_v.at[pl.ds(0, _GB)])
        pltpu.sync_copy(ew_hbm.at[wg], ew_v.at[pl.ds(0, _GB)])
        plsc.subcore_barrier()
        pltpu.async_copy(h_hbm.at[src_v.at[0]], rows0, sg0)

        def scale(rows_ref, row):
            rf = jnp.full((16,), row, jnp.int32)

            def sbody(r, _):
                ewb = plsc.load_gather(
                    ew_v, [rf, jnp.full((16,), r, jnp.int32)])
                for kk in range(_H // 16):
                    sl = pl.ds(kk * 16, 16)
                    rows_ref[r, sl] = rows_ref[r, sl] * ewb
                return 0

            lax.fori_loop(0, _K, sbody, 0)

        def chunk(g, p, jj, b):
            j = g * _GB + jj
            row = p * _GB + jj
            sg_self, sg_other = (sg0, sg1) if b == 0 else (sg1, sg0)
            rows_self, rows_other = (rows0, rows1) if b == 0 else (rows1, rows0)
            pltpu.make_async_copy(h_hbm.at[src_v.at[row]],
                                  rows_self, sg_self).wait()

            @pl.when(j + 1 < nchunk)
            def _():
                @pl.when(jj + 1 < _GB)
                def _():
                    pltpu.async_copy(h_hbm.at[src_v.at[row + 1]],
                                     rows_other, sg_other)

                @pl.when(jj + 1 == _GB)
                def _():
                    q = 1 - p
                    pltpu.make_async_copy(src_hbm.at[wg],
                                          src_v.at[pl.ds(0, _GB)], si).wait()
                    pltpu.make_async_copy(dst_hbm.at[wg],
                                          dst_v.at[pl.ds(0, _GB)], si).wait()
                    pltpu.make_async_copy(ew_hbm.at[wg],
                                          ew_v.at[pl.ds(0, _GB)], si).wait()
                    pltpu.async_copy(h_hbm.at[src_v.at[q * _GB]],
                                     rows_other, sg_other)

            scale(rows_self, row)
            pltpu.sync_copy(rows_self, acc.at[dst_v.at[row]], add=True)

        def pair_body(t, g, p):
            @pl.when(jnp.logical_and(t == 1, g + 1 < ng))
            def _():
                q = 1 - p
                qs = pl.ds(q * _GB, _GB)
                pltpu.async_copy(src_hbm.at[wg + g + 1], src_v.at[qs], si)
                pltpu.async_copy(dst_hbm.at[wg + g + 1], dst_v.at[qs], si)
                pltpu.async_copy(ew_hbm.at[wg + g + 1], ew_v.at[qs], si)

            chunk(g, p, 2 * t, 0)
            chunk(g, p, 2 * t + 1, 1)

        def group_body(g, _):
            p = g % 2

            def inner(t, _):
                pair_body(t, g, p)
                return 0

            lax.fori_loop(0, _GB // 2, inner, 0)
            return 0

        lax.fori_loop(0, ng, group_body, 0)
        plsc.subcore_barrier()
        pltpu.sync_copy(acc.at[pl.ds(s * _RPS, _RPS)],
                        out_hbm.at[c, pl.ds(s * _RPS, _RPS)])

    return agg


@functools.cache
def _build_pool_kernel():
    return functools.partial(
        pl.kernel,
        out_type=jax.ShapeDtypeStruct((_B, 3 * _H), jnp.float32),
        mesh=_sc_mesh(),
        scratch_types=[
            pltpu.VMEM((_TPB,), jnp.int32),
            pltpu.VMEM((_TPB, _H), jnp.float32),
            pltpu.VMEM((_SEG, 3 * _H), jnp.float32),
            pltpu.SemaphoreType.DMA,
        ],
        compiler_params=_SC_PARAMS,
    )(_pool_body)


def _pool_body(h_hbm, bat_hbm, z_hbm, idx_v, rows_v, out_v, sem):
    c = lax.axis_index("c")
    s = lax.axis_index("s")
    w = s * _NC + c
    pltpu.sync_copy(bat_hbm.at[w], idx_v)
    pltpu.async_copy(h_hbm.at[idx_v], rows_v, sem).wait()
    inv_s = jnp.float32(1.0 / _S)
    for t in range(_SEG):
        for kk in range(_H // 16):
            sl = pl.ds(kk * 16, 16)
            v0 = rows_v[t * _S, sl]

            def red_body(rr, carry):
                sm, mn, mx = carry
                v = rows_v[t * _S + rr, sl]
                return (sm + v, jnp.minimum(mn, v), jnp.maximum(mx, v))

            sm, mn, mx = lax.fori_loop(1, _S, red_body, (v0, v0, v0))
            out_v[t, pl.ds(kk * 16, 16)] = sm * inv_s
            out_v[t, pl.ds(_H + kk * 16, 16)] = mn
            out_v[t, pl.ds(2 * _H + kk * 16, 16)] = mx
    pltpu.sync_copy(out_v, z_hbm.at[pl.ds(w * _SEG, _SEG)])


_BR = 1024


def _build_dense_layer(with_f):
    def body(*refs):
        if with_f:
            h_ref, a0_ref, a1_ref, af_ref, w_ref, wf_ref, b_ref, o_ref = refs
        else:
            h_ref, a0_ref, a1_ref, w_ref, b_ref, o_ref = refs
        acc = h_ref[...] + a0_ref[...] + a1_ref[...]
        out = jnp.dot(acc, w_ref[...], preferred_element_type=jnp.float32)
        if with_f:
            fsum = jnp.sum(af_ref[...], axis=0)
            out = out + fsum[:, None] * wf_ref[...]
        o_ref[...] = jnp.maximum(out + b_ref[...], 0.0)

    row_specs = [pl.BlockSpec((_BR, _H), lambda i: (i, 0))] * 3
    if with_f:
        in_specs = row_specs + [
            pl.BlockSpec((_NW + 1, _BR), lambda i: (0, i)),
            pl.BlockSpec((_H, _H), lambda i: (0, 0)),
            pl.BlockSpec((1, _H), lambda i: (0, 0)),
            pl.BlockSpec((1, _H), lambda i: (0, 0)),
        ]
    else:
        in_specs = row_specs + [
            pl.BlockSpec((_H, _H), lambda i: (0, 0)),
            pl.BlockSpec((1, _H), lambda i: (0, 0)),
        ]
    return pl.pallas_call(
        body,
        grid=(_NP // _BR,),
        in_specs=in_specs,
        out_specs=pl.BlockSpec((_BR, _H), lambda i: (i, 0)),
        out_shape=jax.ShapeDtypeStruct((_NP, _H), jnp.float32),
    )


_dense0 = _build_dense_layer(True)
_dense = _build_dense_layer(False)


def _ln(v, g, b, eps=1e-5):
    mu = jnp.mean(v, axis=-1, keepdims=True)
    var = jnp.mean((v - mu) ** 2, axis=-1, keepdims=True)
    return (v - mu) / jnp.sqrt(var + eps) * g + b


def _readout_body(z_ref, y_ref, r0w, r0b, g1r, be1r, r1w, r1b, g2r, be2r,
                  r2wt, r2b, o_ref):
    z = jnp.dot(z_ref[...], r0w[...], preferred_element_type=jnp.float32)
    z = jnp.maximum(z + r0b[...], 0.0)
    z = _ln(z, g1r[...], be1r[...])
    z = jnp.dot(z, r1w[...], preferred_element_type=jnp.float32) + r1b[...]
    z = _ln(z, g2r[...], be2r[...])
    z = jnp.maximum(z, 0.0)
    logits = jnp.sum(z * r2wt[...], axis=1, keepdims=True) + r2b[...]
    y = y_ref[...]
    loss = (jnp.maximum(logits, 0.0) - logits * y
            + jnp.log(1.0 + jnp.exp(-jnp.abs(logits))))
    o_ref[...] = jnp.mean(loss).reshape(1, 1)


_readout = pl.pallas_call(
    _readout_body,
    out_shape=jax.ShapeDtypeStruct((1, 1), jnp.float32),
)


def kernel(x, ei, ew, batches, labels, W0, b0, W1, b1, W2, b2,
           R0W, R0b, g1, be1, R1W, R1b, g2, be2, R2W, R2b):
    src = ei[0].astype(jnp.int32)
    dst = ei[1].astype(jnp.int32)
    pad = _EPAD - _E
    srcp = jnp.concatenate([src, jnp.zeros((pad,), jnp.int32)])
    dstp = jnp.concatenate([dst, jnp.zeros((pad,), jnp.int32)])
    ewp = jnp.concatenate([ew, jnp.zeros((pad,), jnp.float32)])
    srcp = srcp.reshape(_NW * _NG, _GB, _K)
    dstp = dstp.reshape(_NW * _NG, _GB, _K)
    ewp = ewp.reshape(_NW * _NG, _GB, _K)

    xp = jnp.concatenate([x, jnp.zeros((_NP - _N, _D), jnp.float32)], axis=0)

    af = _build_prep_kernel()(batches.astype(jnp.int32),
                              srcp.reshape(_NW, _NCHUNK, _K),
                              dstp.reshape(_NW, _NCHUNK, _K),
                              ewp.reshape(_NW, _NCHUNK, _K))
    agg = _build_agg_kernel()
    a0 = agg(xp, srcp, dstp, ewp)
    h1 = _dense0(xp, a0[0], a0[1], af, W0[:_D], W0[_D:_D + 1],
                 b0.reshape(1, _H))
    a1 = agg(h1, srcp, dstp, ewp)
    h2 = _dense(h1, a1[0], a1[1], W1, b1.reshape(1, _H))
    a2 = agg(h2, srcp, dstp, ewp)
    h3 = _dense(h2, a2[0], a2[1], W2, b2.reshape(1, _H))

    z = _build_pool_kernel()(h3, batches.astype(jnp.int32).reshape(_NW, _TPB))

    loss = _readout(
        z, labels.astype(jnp.float32), R0W, R0b.reshape(1, _H),
        g1.reshape(1, _H), be1.reshape(1, _H), R1W, R1b.reshape(1, _H),
        g2.reshape(1, _H), be2.reshape(1, _H), R2W.reshape(1, _H),
        R2b.reshape(1, 1))
    return loss[0, 0]


# speedup vs baseline: 1.0156x; 1.0133x over previous
import functools

import jax
import jax.numpy as jnp
from jax import lax
from jax.experimental import pallas as pl
from jax.experimental.pallas import tpu as pltpu
from jax.experimental.pallas import tpu_sc as plsc

_N = 10000
_E = 320000
_D = 128
_H = 128
_B = 256
_S = 16

_NC = 2
_NS = 16
_NW = _NC * _NS
_K = 128
_NCHUNK = 80
_EPAD = _NW * _NCHUNK * _K
_GB = 8
_NG = _NCHUNK // _GB
_NGT = 2 * _NG
_G0 = 15
_NP = 10240
_RPS = _NP // _NS
_C0 = 144

_TPB = (_B * _S) // _NW
_SEG = _TPB // _S
_FPT = 640


def _sc_mesh():
    return plsc.VectorSubcoreMesh(core_axis_name="c", subcore_axis_name="s",
                                  num_cores=_NC, num_subcores=_NS)


_SC_PARAMS = pltpu.CompilerParams(needs_layout_passes=False)


@functools.cache
def _build_prep_kernel():
    return functools.partial(
        pl.kernel,
        out_type=jax.ShapeDtypeStruct((_NW + 1, _NP), jnp.float32),
        mesh=_sc_mesh(),
        scratch_types=[
            pltpu.VMEM((_B, _S), jnp.int32),
            pltpu.VMEM((_NCHUNK, _K), jnp.int32),
            pltpu.VMEM((_NCHUNK, _K), jnp.int32),
            pltpu.VMEM((_NCHUNK, _K), jnp.float32),
            pltpu.VMEM((1, _NP), jnp.float32),
            pltpu.VMEM((1, _NP), jnp.float32),
        ],
        compiler_params=_SC_PARAMS,
    )(_prep_body)


def _prep_body(bat_hbm, src_hbm, dst_hbm, ew_hbm, aggf_hbm,
               bat_v, src_v, dst_v, ew_v, f_v, aggf_v):
    c = lax.axis_index("c")
    s = lax.axis_index("s")
    w = s * _NC + c
    pltpu.sync_copy(bat_hbm, bat_v)
    pltpu.sync_copy(src_hbm.at[w], src_v)
    pltpu.sync_copy(dst_hbm.at[w], dst_v)
    pltpu.sync_copy(ew_hbm.at[w], ew_v)
    zeros16 = jnp.zeros((16,), jnp.float32)

    def zero_body(i, _):
        f_v[0, pl.ds(i * 16, 16)] = zeros16
        aggf_v[0, pl.ds(i * 16, 16)] = zeros16
        return 0

    lax.fori_loop(0, _NP // 16, zero_body, 0)
    ones = jnp.ones((16,), jnp.float32)
    z16 = jnp.zeros((16,), jnp.int32)

    def scat_body(i, _):
        plsc.store_scatter(f_v, [z16, bat_v[i, :]], ones)
        return 0

    lax.fori_loop(0, _B, scat_body, 0)

    def grp_body(i, _):
        j = i // (_K // 16)
        g = i % (_K // 16)
        sl = pl.ds(g * 16, 16)
        sv = src_v[j, sl]
        dv = dst_v[j, sl]
        ev = ew_v[j, sl]
        fv = plsc.load_gather(f_v, [z16, sv])
        plsc.addupdate_scatter(aggf_v, [z16, dv], fv * ev)
        return 0

    lax.fori_loop(0, _NCHUNK * (_K // 16), grp_body, 0)
    pltpu.sync_copy(aggf_v, aggf_hbm.at[pl.ds(w, 1)])

    @pl.when(jnp.logical_and(c == 0, s == 0))
    def _():
        pltpu.sync_copy(f_v, aggf_hbm.at[pl.ds(_NW, 1)])


@functools.cache
def _build_agg_kernel():
    @functools.partial(
        pl.kernel,
        out_type=jax.ShapeDtypeStruct((_NC, _NP, _H), jnp.float32),
        mesh=_sc_mesh(),
        scratch_types=[
            pltpu.VMEM((2 * _GB, _K), jnp.int32),
            pltpu.VMEM((2 * _GB, _K), jnp.int32),
            pltpu.VMEM((2 * _GB, _K), jnp.float32),
            pltpu.VMEM((_K, _H), jnp.float32),
            pltpu.VMEM((_K, _H), jnp.float32),
            pltpu.VMEM_SHARED((_NP, _H), jnp.float32),
            pltpu.SemaphoreType.DMA,
            pltpu.SemaphoreType.DMA,
            pltpu.SemaphoreType.DMA,
        ],
        compiler_params=_SC_PARAMS,
    )
    def agg(h_hbm, src_hbm, dst_hbm, ew_hbm, out_hbm,
            src_v, dst_v, ew_v, rows0, rows1, acc, sg0, sg1, si):
        c = lax.axis_index("c")
        s = lax.axis_index("s")
        ng = jnp.where(c == 0, _G0, _NGT - _G0)
        nchunk = ng * _GB
        wg = s * _NGT + jnp.where(c == 0, 0, _G0)
        zeros16 = jnp.zeros((16,), jnp.float32)

        def zbody(i, _):
            rows0[i // (_H // 16), pl.ds((i % (_H // 16)) * 16, 16)] = zeros16
            return 0

        lax.fori_loop(0, _K * (_H // 16), zbody, 0)
        for zz in range(_RPS // _K):
            pltpu.sync_copy(rows0, acc.at[pl.ds(s * _RPS + zz * _K, _K)])
        pltpu.sync_copy(src_hbm.at[wg], src_v.at[pl.ds(0, _GB)])
        pltpu.sync_copy(dst_hbm.at[wg], dst_v.at[pl.ds(0, _GB)])
        pltpu.sync_copy(ew_hbm.at[wg], ew_v.at[pl.ds(0, _GB)])
        plsc.subcore_barrier()
        pltpu.async_copy(h_hbm.at[src_v.at[0]], rows0, sg0)

        def scale(rows_ref, row):
            rf = jnp.full((16,), row, jnp.int32)

            def sbody(r, _):
                ewb = plsc.load_gather(
                    ew_v, [rf, jnp.full((16,), r, jnp.int32)])
                for kk in range(_H // 16):
                    sl = pl.ds(kk * 16, 16)
                    rows_ref[r, sl] = rows_ref[r, sl] * ewb
                return 0

            lax.fori_loop(0, _K, sbody, 0, unroll=4)

        def chunk(g, p, jj, b):
            j = g * _GB + jj
            row = p * _GB + jj
            sg_self, sg_other = (sg0, sg1) if b == 0 else (sg1, sg0)
            rows_self, rows_other = (rows0, rows1) if b == 0 else (rows1, rows0)
            pltpu.make_async_copy(h_hbm.at[src_v.at[row]],
                                  rows_self, sg_self).wait()

            @pl.when(j + 1 < nchunk)
            def _():
                @pl.when(jj + 1 < _GB)
                def _():
                    pltpu.async_copy(h_hbm.at[src_v.at[row + 1]],
                                     rows_other, sg_other)

                @pl.when(jj + 1 == _GB)
                def _():
                    q = 1 - p
                    pltpu.make_async_copy(src_hbm.at[wg],
                                          src_v.at[pl.ds(0, _GB)], si).wait()
                    pltpu.make_async_copy(dst_hbm.at[wg],
                                          dst_v.at[pl.ds(0, _GB)], si).wait()
                    pltpu.make_async_copy(ew_hbm.at[wg],
                                          ew_v.at[pl.ds(0, _GB)], si).wait()
                    pltpu.async_copy(h_hbm.at[src_v.at[q * _GB]],
                                     rows_other, sg_other)

            scale(rows_self, row)
            pltpu.sync_copy(rows_self, acc.at[dst_v.at[row]], add=True)

        def pair_body(t, g, p):
            @pl.when(jnp.logical_and(t == 1, g + 1 < ng))
            def _():
                q = 1 - p
                qs = pl.ds(q * _GB, _GB)
                pltpu.async_copy(src_hbm.at[wg + g + 1], src_v.at[qs], si)
                pltpu.async_copy(dst_hbm.at[wg + g + 1], dst_v.at[qs], si)
                pltpu.async_copy(ew_hbm.at[wg + g + 1], ew_v.at[qs], si)

            chunk(g, p, 2 * t, 0)
            chunk(g, p, 2 * t + 1, 1)

        def group_body(g, _):
            p = g % 2

            def inner(t, _):
                pair_body(t, g, p)
                return 0

            lax.fori_loop(0, _GB // 2, inner, 0)
            return 0

        lax.fori_loop(0, ng, group_body, 0)
        plsc.subcore_barrier()
        pltpu.sync_copy(acc.at[pl.ds(s * _RPS, _RPS)],
                        out_hbm.at[c, pl.ds(s * _RPS, _RPS)])

    return agg


@functools.cache
def _build_pool_kernel():
    return functools.partial(
        pl.kernel,
        out_type=jax.ShapeDtypeStruct((_B, 3 * _H), jnp.float32),
        mesh=_sc_mesh(),
        scratch_types=[
            pltpu.VMEM((_TPB,), jnp.int32),
            pltpu.VMEM((_TPB, _H), jnp.float32),
            pltpu.VMEM((_SEG, 3 * _H), jnp.float32),
            pltpu.SemaphoreType.DMA,
        ],
        compiler_params=_SC_PARAMS,
    )(_pool_body)


def _pool_body(h_hbm, bat_hbm, z_hbm, idx_v, rows_v, out_v, sem):
    c = lax.axis_index("c")
    s = lax.axis_index("s")
    w = s * _NC + c
    pltpu.sync_copy(bat_hbm.at[w], idx_v)
    pltpu.async_copy(h_hbm.at[idx_v], rows_v, sem).wait()
    inv_s = jnp.float32(1.0 / _S)
    for t in range(_SEG):
        for kk in range(_H // 16):
            sl = pl.ds(kk * 16, 16)
            v0 = rows_v[t * _S, sl]

            def red_body(rr, carry):
                sm, mn, mx = carry
                v = rows_v[t * _S + rr, sl]
                return (sm + v, jnp.minimum(mn, v), jnp.maximum(mx, v))

            sm, mn, mx = lax.fori_loop(1, _S, red_body, (v0, v0, v0))
            out_v[t, pl.ds(kk * 16, 16)] = sm * inv_s
            out_v[t, pl.ds(_H + kk * 16, 16)] = mn
            out_v[t, pl.ds(2 * _H + kk * 16, 16)] = mx
    pltpu.sync_copy(out_v, z_hbm.at[pl.ds(w * _SEG, _SEG)])


_BR = 1024


def _build_dense_layer(with_f):
    def body(*refs):
        if with_f:
            h_ref, a0_ref, a1_ref, af_ref, w_ref, wf_ref, b_ref, o_ref = refs
        else:
            h_ref, a0_ref, a1_ref, w_ref, b_ref, o_ref = refs
        acc = h_ref[...] + a0_ref[...] + a1_ref[...]
        out = jnp.dot(acc, w_ref[...], preferred_element_type=jnp.float32)
        if with_f:
            fsum = jnp.sum(af_ref[...], axis=0)
            out = out + fsum[:, None] * wf_ref[...]
        o_ref[...] = jnp.maximum(out + b_ref[...], 0.0)

    row_specs = [pl.BlockSpec((_BR, _H), lambda i: (i, 0))] * 3
    if with_f:
        in_specs = row_specs + [
            pl.BlockSpec((_NW + 1, _BR), lambda i: (0, i)),
            pl.BlockSpec((_H, _H), lambda i: (0, 0)),
            pl.BlockSpec((1, _H), lambda i: (0, 0)),
            pl.BlockSpec((1, _H), lambda i: (0, 0)),
        ]
    else:
        in_specs = row_specs + [
            pl.BlockSpec((_H, _H), lambda i: (0, 0)),
            pl.BlockSpec((1, _H), lambda i: (0, 0)),
        ]
    return pl.pallas_call(
        body,
        grid=(_NP // _BR,),
        in_specs=in_specs,
        out_specs=pl.BlockSpec((_BR, _H), lambda i: (i, 0)),
        out_shape=jax.ShapeDtypeStruct((_NP, _H), jnp.float32),
    )


_dense0 = _build_dense_layer(True)
_dense = _build_dense_layer(False)


def _ln(v, g, b, eps=1e-5):
    mu = jnp.mean(v, axis=-1, keepdims=True)
    var = jnp.mean((v - mu) ** 2, axis=-1, keepdims=True)
    return (v - mu) / jnp.sqrt(var + eps) * g + b


def _readout_body(z_ref, y_ref, r0w, r0b, g1r, be1r, r1w, r1b, g2r, be2r,
                  r2wt, r2b, o_ref):
    z = jnp.dot(z_ref[...], r0w[...], preferred_element_type=jnp.float32)
    z = jnp.maximum(z + r0b[...], 0.0)
    z = _ln(z, g1r[...], be1r[...])
    z = jnp.dot(z, r1w[...], preferred_element_type=jnp.float32) + r1b[...]
    z = _ln(z, g2r[...], be2r[...])
    z = jnp.maximum(z, 0.0)
    logits = jnp.sum(z * r2wt[...], axis=1, keepdims=True) + r2b[...]
    y = y_ref[...]
    loss = (jnp.maximum(logits, 0.0) - logits * y
            + jnp.log(1.0 + jnp.exp(-jnp.abs(logits))))
    o_ref[...] = jnp.mean(loss).reshape(1, 1)


_readout = pl.pallas_call(
    _readout_body,
    out_shape=jax.ShapeDtypeStruct((1, 1), jnp.float32),
)


def kernel(x, ei, ew, batches, labels, W0, b0, W1, b1, W2, b2,
           R0W, R0b, g1, be1, R1W, R1b, g2, be2, R2W, R2b):
    src = ei[0].astype(jnp.int32)
    dst = ei[1].astype(jnp.int32)
    pad = _EPAD - _E
    srcp = jnp.concatenate([src, jnp.zeros((pad,), jnp.int32)])
    dstp = jnp.concatenate([dst, jnp.zeros((pad,), jnp.int32)])
    ewp = jnp.concatenate([ew, jnp.zeros((pad,), jnp.float32)])
    srcp = srcp.reshape(_NW * _NG, _GB, _K)
    dstp = dstp.reshape(_NW * _NG, _GB, _K)
    ewp = ewp.reshape(_NW * _NG, _GB, _K)

    xp = jnp.concatenate([x, jnp.zeros((_NP - _N, _D), jnp.float32)], axis=0)

    af = _build_prep_kernel()(batches.astype(jnp.int32),
                              srcp.reshape(_NW, _NCHUNK, _K),
                              dstp.reshape(_NW, _NCHUNK, _K),
                              ewp.reshape(_NW, _NCHUNK, _K))
    agg = _build_agg_kernel()
    a0 = agg(xp, srcp, dstp, ewp)
    h1 = _dense0(xp, a0[0], a0[1], af, W0[:_D], W0[_D:_D + 1],
                 b0.reshape(1, _H))
    a1 = agg(h1, srcp, dstp, ewp)
    h2 = _dense(h1, a1[0], a1[1], W1, b1.reshape(1, _H))
    a2 = agg(h2, srcp, dstp, ewp)
    h3 = _dense(h2, a2[0], a2[1], W2, b2.reshape(1, _H))

    z = _build_pool_kernel()(h3, batches.astype(jnp.int32).reshape(_NW, _TPB))

    loss = _readout(
        z, labels.astype(jnp.float32), R0W, R0b.reshape(1, _H),
        g1.reshape(1, _H), be1.reshape(1, _H), R1W, R1b.reshape(1, _H),
        g2.reshape(1, _H), be2.reshape(1, _H), R2W.reshape(1, _H),
        R2b.reshape(1, 1))
    return loss[0, 0]
